# Initial kernel scaffold; baseline (speedup 1.0000x reference)
import jax
import jax.numpy as jnp
from jax.experimental import pallas as pl


def kernel(z_e_x, codebook):
    raise NotImplementedError("write your pallas kernel here")


# speedup vs baseline: 1.0179x; 1.0179x over previous
import jax
import jax.numpy as jnp
from jax.experimental import pallas as pl

K = 8192
D = 256
BK = 512
BM = 1024
NTILES = K // BK
SEG = 4096
TILES_PER_SEG = SEG // BK


def _vq_body(xt_ref, cb_ref, csq_ref, xsq_ref, out_ref):
    xt = xt_ref[...]
    xsq = xsq_ref[0]
    seg_v = [None, None]
    seg_i = [None, None]

    for t in range(NTILES):
        s = t // TILES_PER_SEG
        c_t = cb_ref[pl.ds(t * BK, BK), :]
        dots = jax.lax.dot_general(
            c_t, xt, dimension_numbers=(((1,), (0,)), ((), ())),
            preferred_element_type=jnp.float32)
        csq_t = csq_ref[pl.ds(t * BK, BK), 0:1]
        scores = (csq_t + xsq) + 2.0 * dots
        gk = t * BK + jax.lax.broadcasted_iota(jnp.int32, (BK, BM), 0)
        tmin = jnp.min(scores, axis=0, keepdims=True)
        tidx = jnp.min(jnp.where(scores == tmin, gk, K),
                       axis=0, keepdims=True)
        if seg_v[s] is None:
            seg_v[s], seg_i[s] = tmin, tidx
        else:
            upd = tmin < seg_v[s]
            seg_v[s] = jnp.where(upd, tmin, seg_v[s])
            seg_i[s] = jnp.where(upd, tidx, seg_i[s])

    acc_v = seg_v[0].astype(jnp.bfloat16).astype(jnp.float32)
    upd = seg_v[1] < acc_v
    out_ref[0] = jnp.where(upd, seg_i[1], seg_i[0])


@jax.jit
def kernel(z_e_x, codebook):
    Bv, Nv, Dv = z_e_x.shape
    flat = z_e_x.reshape(Bv * Nv, Dv)
    csq = jnp.sum(codebook * codebook, axis=1)
    xsq = jnp.sum(flat * flat, axis=1, keepdims=True)
    csq_b = jnp.broadcast_to(csq[:, None], (K, 128))
    xsq_b = xsq.reshape(Bv, 1, Nv)
    xt_bf16 = flat.T.astype(jnp.bfloat16)
    cb_bf16 = codebook.astype(jnp.bfloat16)

    out = pl.pallas_call(
        _vq_body,
        grid=(Bv * Nv // BM,),
        in_specs=[
            pl.BlockSpec((Dv, BM), lambda i: (0, i)),
            pl.BlockSpec((K, Dv), lambda i: (0, 0)),
            pl.BlockSpec((K, 128), lambda i: (0, 0)),
            pl.BlockSpec((1, 1, BM), lambda i: (i, 0, 0)),
        ],
        out_specs=pl.BlockSpec((1, 1, BM), lambda i: (i, 0, 0)),
        out_shape=jax.ShapeDtypeStruct((Bv * Nv // BM, 1, BM), jnp.int32),
    )(xt_bf16, cb_bf16, csq_b, xsq_b)
    return out.reshape(Bv, Nv)


# speedup vs baseline: 1.5354x; 1.5083x over previous
import jax
import jax.numpy as jnp
from jax.experimental import pallas as pl

K = 8192
D = 256
BK = 512
BM = 1024
NTILES = K // BK
RB = BK // 8
SEG = 4096
TILES_PER_SEG = SEG // BK
NCARRY = 4
BIG = jnp.float32(jnp.inf)


def _combine_pairs(v, i, va, ia):
    take = (va < v) | ((va == v) & (ia < i))
    return jnp.where(take, va, v), jnp.where(take, ia, i)


def _vq_body(xt_ref, cb_ref, csq_ref, xsq_ref, out_ref):
    xt = xt_ref[...]
    xsq_h = xsq_ref[0]
    iota_s = jax.lax.broadcasted_iota(jnp.int32, (8, BM), 0)

    half_v = [None, None]
    half_k = [None, None]
    for h in range(2):
        cv = [None] * NCARRY
        ci = [None] * NCARRY
        for tl in range(TILES_PER_SEG):
            t = h * TILES_PER_SEG + tl
            c_t = cb_ref[pl.ds(t * BK, BK), :]
            dots = jax.lax.dot_general(
                c_t, xt, dimension_numbers=(((1,), (0,)), ((), ())),
                preferred_element_type=jnp.float32)
            csq_h = csq_ref[pl.ds(t * BK, BK), 0:1]
            s = (csq_h + xsq_h) + dots
            s3 = s.reshape(RB, 8, BM)
            for r in range(RB):
                rb = tl * RB + r
                j = rb % NCARRY
                blk = s3[r]
                if cv[j] is None:
                    cv[j] = blk
                    ci[j] = jnp.full((8, BM), rb, jnp.int32)
                else:
                    m = blk < cv[j]
                    cv[j] = jnp.where(m, blk, cv[j])
                    ci[j] = jnp.where(m, rb, ci[j])
        v, i = cv[0], ci[0]
        for j in range(1, NCARRY):
            v, i = _combine_pairs(v, i, cv[j], ci[j])
        kk = i * 8 + iota_s + (h * SEG)
        v1, k1 = _combine_pairs(v[0:4], kk[0:4], v[4:8], kk[4:8])
        v2, k2 = _combine_pairs(v1[0:2], k1[0:2], v1[2:4], k1[2:4])
        half_v[h], half_k[h] = _combine_pairs(v2[0:1], k2[0:1], v2[1:2], k2[1:2])

    acc = half_v[0].astype(jnp.bfloat16).astype(jnp.float32)
    upd = half_v[1] < acc
    out_ref[0] = jnp.where(upd, half_k[1], half_k[0])


@jax.jit
def kernel(z_e_x, codebook):
    Bv, Nv, Dv = z_e_x.shape
    flat = z_e_x.reshape(Bv * Nv, Dv)
    csq = jnp.sum(codebook * codebook, axis=1) * 0.5
    xsq = jnp.sum(flat * flat, axis=1, keepdims=True) * 0.5
    csq_b = jnp.broadcast_to(csq[:, None], (K, 128))
    xsq_b = xsq.reshape(Bv, 1, Nv)
    xt_bf16 = flat.T.astype(jnp.bfloat16)
    cb_bf16 = codebook.astype(jnp.bfloat16)

    out = pl.pallas_call(
        _vq_body,
        grid=(Bv * Nv // BM,),
        in_specs=[
            pl.BlockSpec((Dv, BM), lambda i: (0, i)),
            pl.BlockSpec((K, Dv), lambda i: (0, 0)),
            pl.BlockSpec((K, 128), lambda i: (0, 0)),
            pl.BlockSpec((1, 1, BM), lambda i: (i, 0, 0)),
        ],
        out_specs=pl.BlockSpec((1, 1, BM), lambda i: (i, 0, 0)),
        out_shape=jax.ShapeDtypeStruct((Bv * Nv // BM, 1, BM), jnp.int32),
    )(xt_bf16, cb_bf16, csq_b, xsq_b)
    return out.reshape(Bv, Nv)


# speedup vs baseline: 1.6243x; 1.0579x over previous
import jax
import jax.numpy as jnp
from jax.experimental import pallas as pl

K = 8192
D = 256
BK = 1024
BM = 1024
NTILES = K // BK
RB = BK // 8
SEG = 4096
TILES_PER_SEG = SEG // BK
NCARRY = 2


def _combine_pairs(v, i, va, ia):
    take = (va < v) | ((va == v) & (ia < i))
    return jnp.where(take, va, v), jnp.where(take, ia, i)


def _vq_body(xt_ref, cb_ref, csq_ref, xsq_ref, out_ref):
    xt = xt_ref[...]
    xsq_h = xsq_ref[0]
    iota_s = jax.lax.broadcasted_iota(jnp.int32, (8, BM), 0)

    half_v = [None, None]
    half_k = [None, None]
    for h in range(2):
        cv = [None] * NCARRY
        ci = [None] * NCARRY
        for tl in range(TILES_PER_SEG):
            t = h * TILES_PER_SEG + tl
            c_t = cb_ref[pl.ds(t * BK, BK), :]
            dots = jax.lax.dot_general(
                c_t, xt, dimension_numbers=(((1,), (0,)), ((), ())),
                preferred_element_type=jnp.float32)
            csq_h = csq_ref[pl.ds(t * BK, BK), 0:1]
            s = (csq_h + xsq_h) + dots
            s3 = s.reshape(RB, 8, BM)
            for r in range(RB):
                rb = tl * RB + r
                j = rb % NCARRY
                blk = s3[r]
                if cv[j] is None:
                    cv[j] = blk
                    ci[j] = jnp.full((8, BM), rb, jnp.int32)
                else:
                    m = blk < cv[j]
                    cv[j] = jnp.where(m, blk, cv[j])
                    ci[j] = jnp.where(m, rb, ci[j])
        v, i = cv[0], ci[0]
        for j in range(1, NCARRY):
            v, i = _combine_pairs(v, i, cv[j], ci[j])
        kk = i * 8 + iota_s + (h * SEG)
        v1, k1 = _combine_pairs(v[0:4], kk[0:4], v[4:8], kk[4:8])
        v2, k2 = _combine_pairs(v1[0:2], k1[0:2], v1[2:4], k1[2:4])
        half_v[h], half_k[h] = _combine_pairs(v2[0:1], k2[0:1], v2[1:2], k2[1:2])

    acc = half_v[0].astype(jnp.bfloat16).astype(jnp.float32)
    upd = half_v[1] < acc
    out_ref[0] = jnp.where(upd, half_k[1], half_k[0])


@jax.jit
def kernel(z_e_x, codebook):
    Bv, Nv, Dv = z_e_x.shape
    flat = z_e_x.reshape(Bv * Nv, Dv)
    csq = jnp.sum(codebook * codebook, axis=1) * 0.5
    xsq = jnp.sum(flat * flat, axis=1, keepdims=True) * 0.5
    csq_b = jnp.broadcast_to(csq[:, None], (K, 128))
    xsq_b = xsq.reshape(Bv, 1, Nv)
    xt_bf16 = flat.T.astype(jnp.bfloat16)
    cb_bf16 = codebook.astype(jnp.bfloat16)

    out = pl.pallas_call(
        _vq_body,
        grid=(Bv * Nv // BM,),
        in_specs=[
            pl.BlockSpec((Dv, BM), lambda i: (0, i)),
            pl.BlockSpec((K, Dv), lambda i: (0, 0)),
            pl.BlockSpec((K, 128), lambda i: (0, 0)),
            pl.BlockSpec((1, 1, BM), lambda i: (i, 0, 0)),
        ],
        out_specs=pl.BlockSpec((1, 1, BM), lambda i: (i, 0, 0)),
        out_shape=jax.ShapeDtypeStruct((Bv * Nv // BM, 1, BM), jnp.int32),
    )(xt_bf16, cb_bf16, csq_b, xsq_b)
    return out.reshape(Bv, Nv)


# speedup vs baseline: 1.7082x; 1.0516x over previous
import jax
import jax.numpy as jnp
from jax.experimental import pallas as pl

K = 8192
D = 256
BK = 4096
BM = 1024
NTILES = K // BK
RB = BK // 8
SEG = 4096
TILES_PER_SEG = SEG // BK
NCARRY = 2


def _combine_pairs(v, i, va, ia):
    take = (va < v) | ((va == v) & (ia < i))
    return jnp.where(take, va, v), jnp.where(take, ia, i)


def _vq_body(xt_ref, cb_ref, csq_ref, xsq_ref, out_ref):
    xt = xt_ref[...]
    xsq_h = xsq_ref[0]
    iota_s = jax.lax.broadcasted_iota(jnp.int32, (8, BM), 0)

    half_v = [None, None]
    half_k = [None, None]
    for h in range(2):
        cv = [None] * NCARRY
        ci = [None] * NCARRY
        for tl in range(TILES_PER_SEG):
            t = h * TILES_PER_SEG + tl
            c_t = cb_ref[pl.ds(t * BK, BK), :]
            dots = jax.lax.dot_general(
                c_t, xt, dimension_numbers=(((1,), (0,)), ((), ())),
                preferred_element_type=jnp.float32)
            csq_h = csq_ref[pl.ds(t * BK, BK), 0:1]
            s = (csq_h + xsq_h) + dots
            s3 = s.reshape(RB, 8, BM)
            for r in range(RB):
                rb = tl * RB + r
                j = rb % NCARRY
                blk = s3[r]
                if cv[j] is None:
                    cv[j] = blk
                    ci[j] = jnp.full((8, BM), rb, jnp.int32)
                else:
                    m = blk < cv[j]
                    cv[j] = jnp.where(m, blk, cv[j])
                    ci[j] = jnp.where(m, rb, ci[j])
        v, i = cv[0], ci[0]
        for j in range(1, NCARRY):
            v, i = _combine_pairs(v, i, cv[j], ci[j])
        kk = i * 8 + iota_s + (h * SEG)
        v1, k1 = _combine_pairs(v[0:4], kk[0:4], v[4:8], kk[4:8])
        v2, k2 = _combine_pairs(v1[0:2], k1[0:2], v1[2:4], k1[2:4])
        half_v[h], half_k[h] = _combine_pairs(v2[0:1], k2[0:1], v2[1:2], k2[1:2])

    acc = half_v[0].astype(jnp.bfloat16).astype(jnp.float32)
    upd = half_v[1] < acc
    out_ref[0] = jnp.where(upd, half_k[1], half_k[0])


@jax.jit
def kernel(z_e_x, codebook):
    Bv, Nv, Dv = z_e_x.shape
    flat = z_e_x.reshape(Bv * Nv, Dv)
    csq = jnp.sum(codebook * codebook, axis=1) * 0.5
    xsq = jnp.sum(flat * flat, axis=1, keepdims=True) * 0.5
    csq_b = jnp.broadcast_to(csq[:, None], (K, 128))
    xsq_b = xsq.reshape(Bv * Nv // BM, 1, BM)
    xt_bf16 = flat.T.astype(jnp.bfloat16)
    cb_bf16 = codebook.astype(jnp.bfloat16)

    out = pl.pallas_call(
        _vq_body,
        grid=(Bv * Nv // BM,),
        in_specs=[
            pl.BlockSpec((Dv, BM), lambda i: (0, i)),
            pl.BlockSpec((K, Dv), lambda i: (0, 0)),
            pl.BlockSpec((K, 128), lambda i: (0, 0)),
            pl.BlockSpec((1, 1, BM), lambda i: (i, 0, 0)),
        ],
        out_specs=pl.BlockSpec((1, 1, BM), lambda i: (i, 0, 0)),
        out_shape=jax.ShapeDtypeStruct((Bv * Nv // BM, 1, BM), jnp.int32),
    )(xt_bf16, cb_bf16, csq_b, xsq_b)
    return out.reshape(Bv, Nv)
